# baseline (speedup 1.0000x reference)
import functools

import jax
import jax.numpy as jnp
from jax import lax
from jax.experimental import pallas as pl
from jax.experimental.pallas import tpu as pltpu
from jax.experimental.pallas import tpu_sc as plsc

_RADIUS = 0.2
_K = 32


def _select_body(new_ref, xyz_ref, idx_ref, d2_ref):
    q = new_ref[...]
    x = xyz_ref[...]
    k2 = jnp.sum(x * x, axis=1)[None, :]
    u2 = jnp.sum(q * q, axis=1)[:, None]
    dot = jax.lax.dot_general(q, x, (((1,), (1,)), ((), ())),
                              precision=jax.lax.Precision.DEFAULT,
                              preferred_element_type=jnp.float32)
    d2_ref[...] = u2 + k2 - 2.0 * dot

    Q, N = d2_ref.shape
    L = 128
    G = N // L
    R = 8
    QS = 8
    iota_k = jax.lax.broadcasted_iota(jnp.int32, (QS, _K), 1)
    iota_l = jax.lax.broadcasted_iota(jnp.int32, (QS, L), 1)
    big = jnp.int32(1 << 30)
    inf = jnp.float32(jnp.inf)

    def chunk(qs, _unused):
        qb = qs * QS

        def fold(g, carry):
            vals = list(carry[:R])
            idxs = list(carry[R:])
            v = d2_ref[pl.ds(qb, QS), pl.ds(g * L, L)]
            vi = iota_l + g * L
            for j in range(R):
                c = v < vals[j]
                vals[j], v = jnp.where(c, v, vals[j]), jnp.where(c, vals[j], v)
                idxs[j], vi = jnp.where(c, vi, idxs[j]), jnp.where(c, idxs[j], vi)
            return tuple(vals) + tuple(idxs)

        init = tuple(jnp.full((QS, L), inf) for _ in range(R)) + tuple(
            jnp.full((QS, L), big) for _ in range(R))
        carry = jax.lax.fori_loop(0, G, fold, init, unroll=2)

        def body(i, carry2):
            acc, idx0, vals, idxs = carry2
            vals = list(vals)
            m8 = vals[0]
            for j in range(1, R):
                m8 = jnp.minimum(m8, vals[j])
            m = jnp.min(m8, axis=1, keepdims=True)
            c8 = jnp.where(vals[0] == m, idxs[0], big)
            for j in range(1, R):
                c8 = jnp.minimum(c8, jnp.where(vals[j] == m, idxs[j], big))
            ci = jnp.min(c8, axis=1)
            cib = ci[:, None]
            vals = [jnp.where(idxs[j] == cib, inf, vals[j]) for j in range(R)]
            idx0 = jnp.where(i == 0, ci, idx0)
            inball = jnp.sqrt(jnp.maximum(m[:, 0], 0.0)) <= _RADIUS
            chosen = jnp.where(inball, ci, idx0)
            acc = jnp.where(iota_k == i, chosen[:, None], acc)
            return acc, idx0, tuple(vals), tuple(idxs)

        acc0 = jnp.zeros((QS, _K), jnp.int32)
        idx00 = jnp.zeros((QS,), jnp.int32)
        acc, _, _, _ = jax.lax.fori_loop(
            0, _K, body, (acc0, idx00, carry[:R], carry[R:]))
        idx_ref[pl.ds(qb, QS), :] = acc
        return 0

    jax.lax.fori_loop(0, Q // QS, chunk, 0)


def _knn_idx(new_xyz, xyz):
    B, P, _ = new_xyz.shape
    N = xyz.shape[1]
    Q = min(256, P)
    grid = (B, P // Q)
    return pl.pallas_call(
        _select_body,
        grid=grid,
        in_specs=[
            pl.BlockSpec((None, Q, 3), lambda b, p: (b, p, 0)),
            pl.BlockSpec((None, N, 3), lambda b, p: (b, 0, 0)),
        ],
        out_specs=pl.BlockSpec((None, Q, _K), lambda b, p: (b, p, 0)),
        out_shape=jax.ShapeDtypeStruct((B, P, _K), jnp.int32),
        scratch_shapes=[pltpu.VMEM((Q, N), jnp.float32)],
    )(new_xyz, xyz)


def _sc_gather(G, idx_flat, new_t):
    B, C3, N = G.shape
    PK = idx_flat.shape[1]
    P = new_t.shape[2]
    mesh = plsc.VectorSubcoreMesh(core_axis_name="c", subcore_axis_name="s")
    tiles_per_batch = 32 // B
    n_ch = (C3 + tiles_per_batch - 1) // tiles_per_batch
    niter = PK // 16

    @functools.partial(
        pl.kernel,
        mesh=mesh,
        compiler_params=pltpu.CompilerParams(needs_layout_passes=False),
        out_type=[jax.ShapeDtypeStruct((B, C3, PK), jnp.float32),
                  jax.ShapeDtypeStruct((B, 3, PK), jnp.float32)],
        scratch_types=[
            pltpu.VMEM((PK,), jnp.int32),
            pltpu.VMEM((N,), jnp.float32),
            pltpu.VMEM((PK,), jnp.float32),
            pltpu.VMEM((P,), jnp.float32),
        ],
    )
    def k(g_hbm, idx_hbm, newt_hbm, nf_hbm, gx_hbm, idx_v, row_v, out_v, newp_v):
        wid = lax.axis_index("s") * 2 + lax.axis_index("c")
        b = wid // tiles_per_batch
        t = wid % tiles_per_batch
        pltpu.sync_copy(idx_hbm.at[b], idx_v)
        iota16 = lax.broadcasted_iota(jnp.int32, (16,), 0)

        for kk in range(n_ch):
            c = t + tiles_per_batch * kk

            @pl.when(c < C3)
            def _():
                pltpu.sync_copy(g_hbm.at[b, c], row_v)

                @pl.when(c < 3)
                def _():
                    pltpu.sync_copy(newt_hbm.at[b, c], newp_v)

                is_xyz = (jnp.zeros((16,), jnp.int32) + c) < 3

                def body(j, carry):
                    base = j * 16
                    iv = idx_v[pl.ds(base, 16)]
                    vals = plsc.load_gather(row_v, [iv])
                    pvec = lax.shift_right_logical(iota16 + base, _K.bit_length() - 1)
                    qv = plsc.load_gather(newp_v, [pvec])
                    out_v[pl.ds(base, 16)] = jnp.where(is_xyz, vals - qv, vals)
                    return carry

                lax.fori_loop(0, niter, body, 0)
                pltpu.sync_copy(out_v, nf_hbm.at[b, c])

                @pl.when(c < 3)
                def _():
                    pltpu.sync_copy(out_v, gx_hbm.at[b, c])

    return k(G, idx_flat, new_t)


def kernel(xyz, new_xyz, features):
    B, N, _ = xyz.shape
    P = new_xyz.shape[1]
    C = features.shape[1]
    xyz_t = jnp.transpose(xyz, (0, 2, 1))
    new_t = jnp.transpose(new_xyz, (0, 2, 1))
    idx = _knn_idx(new_xyz, xyz)

    G = jnp.concatenate([xyz_t, features], axis=1)
    nf, gx = _sc_gather(G, idx.reshape(B, P * _K), new_t)
    new_features = nf.reshape(B, 3 + C, P, _K)
    grouped_xyz = gx.reshape(B, 3, P, _K)
    return (new_features, grouped_xyz)


# speedup vs baseline: 2.5989x; 2.5989x over previous
import functools

import jax
import jax.numpy as jnp
from jax import lax
from jax.experimental import pallas as pl
from jax.experimental.pallas import tpu as pltpu
from jax.experimental.pallas import tpu_sc as plsc

_RADIUS = 0.2
_K = 32


def _select_body(new_ref, xyz_ref, idx_ref, d2_ref):
    q = new_ref[...]
    x = xyz_ref[...]
    k2 = jnp.sum(x * x, axis=1)[None, :]
    u2 = jnp.sum(q * q, axis=1)[:, None]
    dot = jax.lax.dot_general(q, x, (((1,), (1,)), ((), ())),
                              precision=jax.lax.Precision.DEFAULT,
                              preferred_element_type=jnp.float32)
    d2_ref[...] = u2 + k2 - 2.0 * dot

    Q, N = d2_ref.shape
    L = 128
    G = N // L
    R = 8
    iota_k = jax.lax.broadcasted_iota(jnp.int32, (Q, _K), 1)
    iota_l = jax.lax.broadcasted_iota(jnp.int32, (Q, L), 1)
    big = jnp.int32(1 << 30)
    inf = jnp.float32(jnp.inf)

    def fold(g, carry):
        vals = list(carry[:R])
        idxs = list(carry[R:])
        v = d2_ref[:, pl.ds(g * L, L)]
        vi = iota_l + g * L
        for j in range(R):
            c = v < vals[j]
            nmin = jnp.minimum(v, vals[j])
            nmax = jnp.maximum(v, vals[j])
            vals[j], v = nmin, nmax
            idxs[j], vi = jnp.where(c, vi, idxs[j]), jnp.where(c, idxs[j], vi)
        return tuple(vals) + tuple(idxs)

    init = tuple(jnp.full((Q, L), inf) for _ in range(R)) + tuple(
        jnp.full((Q, L), big) for _ in range(R))
    carry = jax.lax.fori_loop(0, G, fold, init, unroll=2)

    def body(i, carry2):
        acc, idx0, vals, idxs = carry2
        vals = list(vals)
        m8 = vals[0]
        for j in range(1, R):
            m8 = jnp.minimum(m8, vals[j])
        m = jnp.min(m8, axis=1, keepdims=True)
        c8 = jnp.where(vals[0] == m, idxs[0], big)
        for j in range(1, R):
            c8 = jnp.minimum(c8, jnp.where(vals[j] == m, idxs[j], big))
        ci = jnp.min(c8, axis=1)
        cib = ci[:, None]
        vals = [jnp.where(idxs[j] == cib, inf, vals[j]) for j in range(R)]
        idx0 = jnp.where(i == 0, ci, idx0)
        inball = jnp.sqrt(jnp.maximum(m[:, 0], 0.0)) <= _RADIUS
        chosen = jnp.where(inball, ci, idx0)
        acc = jnp.where(iota_k == i, chosen[:, None], acc)
        return acc, idx0, tuple(vals), tuple(idxs)

    acc0 = jnp.zeros((Q, _K), jnp.int32)
    idx00 = jnp.zeros((Q,), jnp.int32)
    acc, _, _, _ = jax.lax.fori_loop(
        0, _K, body, (acc0, idx00, carry[:R], carry[R:]))
    idx_ref[...] = acc


def _knn_idx(new_xyz, xyz):
    B, P, _ = new_xyz.shape
    N = xyz.shape[1]
    Q = min(32, P)
    grid = (B, P // Q)
    return pl.pallas_call(
        _select_body,
        grid=grid,
        in_specs=[
            pl.BlockSpec((None, Q, 3), lambda b, p: (b, p, 0)),
            pl.BlockSpec((None, N, 3), lambda b, p: (b, 0, 0)),
        ],
        out_specs=pl.BlockSpec((None, Q, _K), lambda b, p: (b, p, 0)),
        out_shape=jax.ShapeDtypeStruct((B, P, _K), jnp.int32),
        scratch_shapes=[pltpu.VMEM((Q, N), jnp.float32)],
    )(new_xyz, xyz)


def _sc_gather(G, idx_flat, new_t):
    B, C3, N = G.shape
    PK = idx_flat.shape[1]
    P = new_t.shape[2]
    mesh = plsc.VectorSubcoreMesh(core_axis_name="c", subcore_axis_name="s")
    tiles_per_batch = 32 // B
    n_ch = (C3 + tiles_per_batch - 1) // tiles_per_batch
    niter = PK // 16

    @functools.partial(
        pl.kernel,
        mesh=mesh,
        compiler_params=pltpu.CompilerParams(needs_layout_passes=False),
        out_type=[jax.ShapeDtypeStruct((B, C3, PK), jnp.float32),
                  jax.ShapeDtypeStruct((B, 3, PK), jnp.float32)],
        scratch_types=[
            pltpu.VMEM((PK,), jnp.int32),
            pltpu.VMEM((N,), jnp.float32),
            pltpu.VMEM((PK,), jnp.float32),
            pltpu.VMEM((P,), jnp.float32),
        ],
    )
    def k(g_hbm, idx_hbm, newt_hbm, nf_hbm, gx_hbm, idx_v, row_v, out_v, newp_v):
        wid = lax.axis_index("s") * 2 + lax.axis_index("c")
        b = wid // tiles_per_batch
        t = wid % tiles_per_batch
        pltpu.sync_copy(idx_hbm.at[b], idx_v)
        iota16 = lax.broadcasted_iota(jnp.int32, (16,), 0)

        for kk in range(n_ch):
            c = t + tiles_per_batch * kk

            @pl.when(c < C3)
            def _():
                pltpu.sync_copy(g_hbm.at[b, c], row_v)

                @pl.when(c < 3)
                def _():
                    pltpu.sync_copy(newt_hbm.at[b, c], newp_v)

                is_xyz = (jnp.zeros((16,), jnp.int32) + c) < 3

                def body(j, carry):
                    base = j * 16
                    iv = idx_v[pl.ds(base, 16)]
                    vals = plsc.load_gather(row_v, [iv])
                    pvec = lax.shift_right_logical(iota16 + base, _K.bit_length() - 1)
                    qv = plsc.load_gather(newp_v, [pvec])
                    out_v[pl.ds(base, 16)] = jnp.where(is_xyz, vals - qv, vals)
                    return carry

                lax.fori_loop(0, niter, body, 0)
                pltpu.sync_copy(out_v, nf_hbm.at[b, c])

                @pl.when(c < 3)
                def _():
                    pltpu.sync_copy(out_v, gx_hbm.at[b, c])

    return k(G, idx_flat, new_t)


def kernel(xyz, new_xyz, features):
    B, N, _ = xyz.shape
    P = new_xyz.shape[1]
    C = features.shape[1]
    xyz_t = jnp.transpose(xyz, (0, 2, 1))
    new_t = jnp.transpose(new_xyz, (0, 2, 1))
    idx = _knn_idx(new_xyz, xyz)

    G = jnp.concatenate([xyz_t, features], axis=1)
    nf, gx = _sc_gather(G, idx.reshape(B, P * _K), new_t)
    new_features = nf.reshape(B, 3 + C, P, _K)
    grouped_xyz = gx.reshape(B, 3, P, _K)
    return (new_features, grouped_xyz)


# speedup vs baseline: 2.9047x; 1.1177x over previous
import functools

import jax
import jax.numpy as jnp
from jax import lax
from jax.experimental import pallas as pl
from jax.experimental.pallas import tpu as pltpu
from jax.experimental.pallas import tpu_sc as plsc

_RADIUS = 0.2
_K = 32


def _k2_body(xyz_ref, k2_ref):
    x = xyz_ref[...]
    k2_ref[...] = jnp.sum(x * x, axis=1)[None, :]


def _select_body(new_ref, xyz_ref, k2_ref, idx_ref, d2_ref):
    q = new_ref[...]
    x = xyz_ref[...]
    k2 = k2_ref[...]
    u2 = jnp.sum(q * q, axis=1)[:, None]
    dot = jax.lax.dot_general(q, x, (((1,), (1,)), ((), ())),
                              precision=jax.lax.Precision.DEFAULT,
                              preferred_element_type=jnp.float32)
    d2_ref[...] = u2 + k2 - 2.0 * dot

    Q, N = d2_ref.shape
    L = 128
    G = N // L
    R = 8
    iota_k = jax.lax.broadcasted_iota(jnp.int32, (Q, _K), 1)
    iota_l = jax.lax.broadcasted_iota(jnp.int32, (Q, L), 1)
    big = jnp.int32(1 << 30)
    inf = jnp.float32(jnp.inf)

    vals = [jnp.full((Q, L), inf) for _ in range(R)]
    idxs = [jnp.full((Q, L), big) for _ in range(R)]
    for g in range(G):
        v = d2_ref[:, g * L:(g + 1) * L]
        vi = iota_l + g * L
        for j in range(R):
            c = v < vals[j]
            nmin = jnp.minimum(v, vals[j])
            nmax = jnp.maximum(v, vals[j])
            vals[j], v = nmin, nmax
            idxs[j], vi = jnp.where(c, vi, idxs[j]), jnp.where(c, idxs[j], vi)
    carry = tuple(vals) + tuple(idxs)

    def body(i, carry2):
        acc, idx0, vals, idxs = carry2
        vals = list(vals)
        m8 = vals[0]
        for j in range(1, R):
            m8 = jnp.minimum(m8, vals[j])
        m = jnp.min(m8, axis=1, keepdims=True)
        c8 = jnp.where(vals[0] == m, idxs[0], big)
        for j in range(1, R):
            c8 = jnp.minimum(c8, jnp.where(vals[j] == m, idxs[j], big))
        ci = jnp.min(c8, axis=1)
        cib = ci[:, None]
        vals = [jnp.where(idxs[j] == cib, inf, vals[j]) for j in range(R)]
        idx0 = jnp.where(i == 0, ci, idx0)
        inball = jnp.sqrt(jnp.maximum(m[:, 0], 0.0)) <= _RADIUS
        chosen = jnp.where(inball, ci, idx0)
        acc = jnp.where(iota_k == i, chosen[:, None], acc)
        return acc, idx0, tuple(vals), tuple(idxs)

    acc0 = jnp.zeros((Q, _K), jnp.int32)
    idx00 = jnp.zeros((Q,), jnp.int32)
    acc, _, _, _ = jax.lax.fori_loop(
        0, _K, body, (acc0, idx00, carry[:R], carry[R:]))
    idx_ref[...] = acc


def _knn_idx(new_xyz, xyz):
    B, P, _ = new_xyz.shape
    N = xyz.shape[1]
    Q = min(32, P)
    grid = (B, P // Q)
    k2 = pl.pallas_call(
        _k2_body,
        grid=(B,),
        in_specs=[pl.BlockSpec((None, N, 3), lambda b: (b, 0, 0))],
        out_specs=pl.BlockSpec((None, 1, N), lambda b: (b, 0, 0)),
        out_shape=jax.ShapeDtypeStruct((B, 1, N), jnp.float32),
    )(xyz)
    return pl.pallas_call(
        _select_body,
        grid=grid,
        in_specs=[
            pl.BlockSpec((None, Q, 3), lambda b, p: (b, p, 0)),
            pl.BlockSpec((None, N, 3), lambda b, p: (b, 0, 0)),
            pl.BlockSpec((None, 1, N), lambda b, p: (b, 0, 0)),
        ],
        out_specs=pl.BlockSpec((None, Q, _K), lambda b, p: (b, p, 0)),
        out_shape=jax.ShapeDtypeStruct((B, P, _K), jnp.int32),
        scratch_shapes=[pltpu.VMEM((Q, N), jnp.float32)],
    )(new_xyz, xyz, k2)


def _sc_gather(G, idx_flat, new_t):
    B, C3, N = G.shape
    PK = idx_flat.shape[1]
    P = new_t.shape[2]
    mesh = plsc.VectorSubcoreMesh(core_axis_name="c", subcore_axis_name="s")
    tiles_per_batch = 32 // B
    n_ch = (C3 + tiles_per_batch - 1) // tiles_per_batch
    niter = PK // 16

    @functools.partial(
        pl.kernel,
        mesh=mesh,
        compiler_params=pltpu.CompilerParams(needs_layout_passes=False),
        out_type=[jax.ShapeDtypeStruct((B, C3, PK), jnp.float32),
                  jax.ShapeDtypeStruct((B, 3, PK), jnp.float32)],
        scratch_types=[
            pltpu.VMEM((PK,), jnp.int32),
            pltpu.VMEM((N,), jnp.float32),
            pltpu.VMEM((PK,), jnp.float32),
            pltpu.VMEM((P,), jnp.float32),
        ],
    )
    def k(g_hbm, idx_hbm, newt_hbm, nf_hbm, gx_hbm, idx_v, row_v, out_v, newp_v):
        wid = lax.axis_index("s") * 2 + lax.axis_index("c")
        b = wid // tiles_per_batch
        t = wid % tiles_per_batch
        pltpu.sync_copy(idx_hbm.at[b], idx_v)
        iota16 = lax.broadcasted_iota(jnp.int32, (16,), 0)

        for kk in range(n_ch):
            c = t + tiles_per_batch * kk

            @pl.when(c < C3)
            def _():
                pltpu.sync_copy(g_hbm.at[b, c], row_v)

                @pl.when(c < 3)
                def _():
                    pltpu.sync_copy(newt_hbm.at[b, c], newp_v)

                is_xyz = (jnp.zeros((16,), jnp.int32) + c) < 3

                def body(j, carry):
                    base = j * 16
                    iv = idx_v[pl.ds(base, 16)]
                    vals = plsc.load_gather(row_v, [iv])
                    pvec = lax.shift_right_logical(iota16 + base, _K.bit_length() - 1)
                    qv = plsc.load_gather(newp_v, [pvec])
                    out_v[pl.ds(base, 16)] = jnp.where(is_xyz, vals - qv, vals)
                    return carry

                lax.fori_loop(0, niter, body, 0)
                pltpu.sync_copy(out_v, nf_hbm.at[b, c])

                @pl.when(c < 3)
                def _():
                    pltpu.sync_copy(out_v, gx_hbm.at[b, c])

    return k(G, idx_flat, new_t)


def kernel(xyz, new_xyz, features):
    B, N, _ = xyz.shape
    P = new_xyz.shape[1]
    C = features.shape[1]
    xyz_t = jnp.transpose(xyz, (0, 2, 1))
    new_t = jnp.transpose(new_xyz, (0, 2, 1))
    idx = _knn_idx(new_xyz, xyz)

    G = jnp.concatenate([xyz_t, features], axis=1)
    nf, gx = _sc_gather(G, idx.reshape(B, P * _K), new_t)
    new_features = nf.reshape(B, 3 + C, P, _K)
    grouped_xyz = gx.reshape(B, 3, P, _K)
    return (new_features, grouped_xyz)


# speedup vs baseline: 5.0650x; 1.7437x over previous
import functools

import jax
import jax.numpy as jnp
from jax import lax
from jax.experimental import pallas as pl
from jax.experimental.pallas import tpu as pltpu
from jax.experimental.pallas import tpu_sc as plsc

_RADIUS = 0.2
_K = 32


def _k2_body(xyz_ref, k2_ref):
    x = xyz_ref[...]
    k2_ref[...] = jnp.sum(x * x, axis=1)[None, :]


def _select_body(new_ref, xyz_ref, k2_ref, idx_ref, d2_ref):
    q = new_ref[...]
    x = xyz_ref[...]
    k2 = k2_ref[...]
    u2 = jnp.sum(q * q, axis=1)[:, None]
    dot = jax.lax.dot_general(q, x, (((1,), (1,)), ((), ())),
                              precision=jax.lax.Precision.DEFAULT,
                              preferred_element_type=jnp.float32)
    d2_ref[...] = u2 + k2 - 2.0 * dot

    Q, N = d2_ref.shape
    L = 128
    G = N // L
    R = 8
    iota_k = jax.lax.broadcasted_iota(jnp.int32, (Q, _K), 1)
    iota_l = jax.lax.broadcasted_iota(jnp.int32, (Q, L), 1)
    big = jnp.int32(1 << 30)
    inf = jnp.float32(jnp.inf)

    vals = [jnp.full((Q, L), inf) for _ in range(R)]
    idxs = [jnp.full((Q, L), big) for _ in range(R)]
    for g in range(G):
        v = d2_ref[:, g * L:(g + 1) * L]
        vi = iota_l + g * L
        for j in range(R):
            c = v < vals[j]
            nmin = jnp.minimum(v, vals[j])
            nmax = jnp.maximum(v, vals[j])
            vals[j], v = nmin, nmax
            idxs[j], vi = jnp.where(c, vi, idxs[j]), jnp.where(c, idxs[j], vi)
    carry = tuple(vals) + tuple(idxs)

    def body(i, carry2):
        acc, idx0, vals, idxs = carry2
        vals = list(vals)
        m8 = vals[0]
        for j in range(1, R):
            m8 = jnp.minimum(m8, vals[j])
        m = jnp.min(m8, axis=1, keepdims=True)
        c8 = jnp.where(vals[0] == m, idxs[0], big)
        for j in range(1, R):
            c8 = jnp.minimum(c8, jnp.where(vals[j] == m, idxs[j], big))
        ci = jnp.min(c8, axis=1)
        cib = ci[:, None]
        vals = [jnp.where(idxs[j] == cib, inf, vals[j]) for j in range(R)]
        idx0 = jnp.where(i == 0, ci, idx0)
        inball = jnp.sqrt(jnp.maximum(m[:, 0], 0.0)) <= _RADIUS
        chosen = jnp.where(inball, ci, idx0)
        acc = jnp.where(iota_k == i, chosen[:, None], acc)
        return acc, idx0, tuple(vals), tuple(idxs)

    acc0 = jnp.zeros((Q, _K), jnp.int32)
    idx00 = jnp.zeros((Q,), jnp.int32)
    acc, _, _, _ = jax.lax.fori_loop(
        0, _K, body, (acc0, idx00, carry[:R], carry[R:]))
    idx_ref[...] = acc


def _knn_idx(new_xyz, xyz):
    B, P, _ = new_xyz.shape
    N = xyz.shape[1]
    Q = min(128, P)
    grid = (B, P // Q)
    k2 = pl.pallas_call(
        _k2_body,
        grid=(B,),
        in_specs=[pl.BlockSpec((None, N, 3), lambda b: (b, 0, 0))],
        out_specs=pl.BlockSpec((None, 1, N), lambda b: (b, 0, 0)),
        out_shape=jax.ShapeDtypeStruct((B, 1, N), jnp.float32),
    )(xyz)
    return pl.pallas_call(
        _select_body,
        grid=grid,
        in_specs=[
            pl.BlockSpec((None, Q, 3), lambda b, p: (b, p, 0)),
            pl.BlockSpec((None, N, 3), lambda b, p: (b, 0, 0)),
            pl.BlockSpec((None, 1, N), lambda b, p: (b, 0, 0)),
        ],
        out_specs=pl.BlockSpec((None, Q, _K), lambda b, p: (b, p, 0)),
        out_shape=jax.ShapeDtypeStruct((B, P, _K), jnp.int32),
        scratch_shapes=[pltpu.VMEM((Q, N), jnp.float32)],
    )(new_xyz, xyz, k2)


def _sc_gather(G, idx_flat, new_t):
    B, C3, N = G.shape
    PK = idx_flat.shape[1]
    P = new_t.shape[2]
    mesh = plsc.VectorSubcoreMesh(core_axis_name="c", subcore_axis_name="s")
    tiles_per_batch = 32 // B
    n_ch = (C3 + tiles_per_batch - 1) // tiles_per_batch
    niter = PK // 16

    @functools.partial(
        pl.kernel,
        mesh=mesh,
        compiler_params=pltpu.CompilerParams(needs_layout_passes=False),
        out_type=[jax.ShapeDtypeStruct((B, C3, PK), jnp.float32),
                  jax.ShapeDtypeStruct((B, 3, PK), jnp.float32)],
        scratch_types=[
            pltpu.VMEM((PK,), jnp.int32),
            pltpu.VMEM((N,), jnp.float32),
            pltpu.VMEM((PK,), jnp.float32),
            pltpu.VMEM((P,), jnp.float32),
        ],
    )
    def k(g_hbm, idx_hbm, newt_hbm, nf_hbm, gx_hbm, idx_v, row_v, out_v, newp_v):
        wid = lax.axis_index("s") * 2 + lax.axis_index("c")
        b = wid // tiles_per_batch
        t = wid % tiles_per_batch
        pltpu.sync_copy(idx_hbm.at[b], idx_v)
        iota16 = lax.broadcasted_iota(jnp.int32, (16,), 0)

        for kk in range(n_ch):
            c = t + tiles_per_batch * kk

            @pl.when(c < C3)
            def _():
                pltpu.sync_copy(g_hbm.at[b, c], row_v)

                @pl.when(c < 3)
                def _():
                    pltpu.sync_copy(newt_hbm.at[b, c], newp_v)

                is_xyz = (jnp.zeros((16,), jnp.int32) + c) < 3

                def body(j, carry):
                    base = j * 16
                    iv = idx_v[pl.ds(base, 16)]
                    vals = plsc.load_gather(row_v, [iv])
                    pvec = lax.shift_right_logical(iota16 + base, _K.bit_length() - 1)
                    qv = plsc.load_gather(newp_v, [pvec])
                    out_v[pl.ds(base, 16)] = jnp.where(is_xyz, vals - qv, vals)
                    return carry

                lax.fori_loop(0, niter, body, 0)
                pltpu.sync_copy(out_v, nf_hbm.at[b, c])

                @pl.when(c < 3)
                def _():
                    pltpu.sync_copy(out_v, gx_hbm.at[b, c])

    return k(G, idx_flat, new_t)


def kernel(xyz, new_xyz, features):
    B, N, _ = xyz.shape
    P = new_xyz.shape[1]
    C = features.shape[1]
    xyz_t = jnp.transpose(xyz, (0, 2, 1))
    new_t = jnp.transpose(new_xyz, (0, 2, 1))
    idx = _knn_idx(new_xyz, xyz)

    G = jnp.concatenate([xyz_t, features], axis=1)
    nf, gx = _sc_gather(G, idx.reshape(B, P * _K), new_t)
    new_features = nf.reshape(B, 3 + C, P, _K)
    grouped_xyz = gx.reshape(B, 3, P, _K)
    return (new_features, grouped_xyz)


# speedup vs baseline: 5.8288x; 1.1508x over previous
import functools

import jax
import jax.numpy as jnp
from jax import lax
from jax.experimental import pallas as pl
from jax.experimental.pallas import tpu as pltpu
from jax.experimental.pallas import tpu_sc as plsc

_RADIUS = 0.2
_K = 32


def _k2_body(xyz_ref, k2_ref):
    x = xyz_ref[...]
    k2_ref[...] = jnp.sum(x * x, axis=1)[None, :]


def _select_body(new_ref, xyz_ref, k2_ref, idx_ref, d2_ref):
    q = new_ref[...]
    x = xyz_ref[...]
    k2 = k2_ref[...]
    u2 = jnp.sum(q * q, axis=1)[:, None]
    dot = jax.lax.dot_general(q, x, (((1,), (1,)), ((), ())),
                              precision=jax.lax.Precision.DEFAULT,
                              preferred_element_type=jnp.float32)
    d2_ref[...] = u2 + k2 - 2.0 * dot

    Q, N = d2_ref.shape
    L = 128
    G = N // L
    R = 8
    iota_k = jax.lax.broadcasted_iota(jnp.int32, (Q, _K), 1)
    iota_l = jax.lax.broadcasted_iota(jnp.int32, (Q, L), 1)
    big = jnp.int32(1 << 30)
    inf = jnp.float32(jnp.inf)

    vals = [jnp.full((Q, L), inf) for _ in range(R)]
    idxs = [jnp.full((Q, L), big) for _ in range(R)]
    for g in range(G):
        v = d2_ref[:, g * L:(g + 1) * L]
        vi = iota_l + g * L
        for j in range(R):
            c = v < vals[j]
            nmin = jnp.minimum(v, vals[j])
            nmax = jnp.maximum(v, vals[j])
            vals[j], v = nmin, nmax
            idxs[j], vi = jnp.where(c, vi, idxs[j]), jnp.where(c, idxs[j], vi)
    carry = tuple(vals) + tuple(idxs)

    def body(i, carry2):
        acc, idx0, vals, idxs = carry2
        vals = list(vals)
        m8 = vals[0]
        for j in range(1, R):
            m8 = jnp.minimum(m8, vals[j])
        m = jnp.min(m8, axis=1, keepdims=True)
        c8 = jnp.where(vals[0] == m, idxs[0], big)
        for j in range(1, R):
            c8 = jnp.minimum(c8, jnp.where(vals[j] == m, idxs[j], big))
        ci = jnp.min(c8, axis=1)
        cib = ci[:, None]
        vals = [jnp.where(idxs[j] == cib, inf, vals[j]) for j in range(R)]
        idx0 = jnp.where(i == 0, ci, idx0)
        inball = jnp.sqrt(jnp.maximum(m[:, 0], 0.0)) <= _RADIUS
        chosen = jnp.where(inball, ci, idx0)
        acc = jnp.where(iota_k == i, chosen[:, None], acc)
        return acc, idx0, tuple(vals), tuple(idxs)

    acc0 = jnp.zeros((Q, _K), jnp.int32)
    idx00 = jnp.zeros((Q,), jnp.int32)
    acc, _, _, _ = jax.lax.fori_loop(
        0, _K, body, (acc0, idx00, carry[:R], carry[R:]))
    idx_ref[...] = acc


def _knn_idx(new_xyz, xyz):
    B, P, _ = new_xyz.shape
    N = xyz.shape[1]
    Q = min(256, P)
    grid = (B, P // Q)
    k2 = pl.pallas_call(
        _k2_body,
        grid=(B,),
        in_specs=[pl.BlockSpec((None, N, 3), lambda b: (b, 0, 0))],
        out_specs=pl.BlockSpec((None, 1, N), lambda b: (b, 0, 0)),
        out_shape=jax.ShapeDtypeStruct((B, 1, N), jnp.float32),
    )(xyz)
    return pl.pallas_call(
        _select_body,
        grid=grid,
        in_specs=[
            pl.BlockSpec((None, Q, 3), lambda b, p: (b, p, 0)),
            pl.BlockSpec((None, N, 3), lambda b, p: (b, 0, 0)),
            pl.BlockSpec((None, 1, N), lambda b, p: (b, 0, 0)),
        ],
        out_specs=pl.BlockSpec((None, Q, _K), lambda b, p: (b, p, 0)),
        out_shape=jax.ShapeDtypeStruct((B, P, _K), jnp.int32),
        scratch_shapes=[pltpu.VMEM((Q, N), jnp.float32)],
    )(new_xyz, xyz, k2)


def _sc_gather(G, idx_flat, new_t):
    B, C3, N = G.shape
    PK = idx_flat.shape[1]
    P = new_t.shape[2]
    mesh = plsc.VectorSubcoreMesh(core_axis_name="c", subcore_axis_name="s")
    tiles_per_batch = 32 // B
    n_ch = (C3 + tiles_per_batch - 1) // tiles_per_batch
    niter = PK // 16

    @functools.partial(
        pl.kernel,
        mesh=mesh,
        compiler_params=pltpu.CompilerParams(needs_layout_passes=False),
        out_type=[jax.ShapeDtypeStruct((B, C3, PK), jnp.float32),
                  jax.ShapeDtypeStruct((B, 3, PK), jnp.float32)],
        scratch_types=[
            pltpu.VMEM((PK,), jnp.int32),
            pltpu.VMEM((N,), jnp.float32),
            pltpu.VMEM((PK,), jnp.float32),
            pltpu.VMEM((P,), jnp.float32),
        ],
    )
    def k(g_hbm, idx_hbm, newt_hbm, nf_hbm, gx_hbm, idx_v, row_v, out_v, newp_v):
        wid = lax.axis_index("s") * 2 + lax.axis_index("c")
        b = wid // tiles_per_batch
        t = wid % tiles_per_batch
        pltpu.sync_copy(idx_hbm.at[b], idx_v)
        iota16 = lax.broadcasted_iota(jnp.int32, (16,), 0)

        for kk in range(n_ch):
            c = t + tiles_per_batch * kk

            @pl.when(c < C3)
            def _():
                pltpu.sync_copy(g_hbm.at[b, c], row_v)

                @pl.when(c < 3)
                def _():
                    pltpu.sync_copy(newt_hbm.at[b, c], newp_v)

                is_xyz = (jnp.zeros((16,), jnp.int32) + c) < 3

                def body(j, carry):
                    base = j * 16
                    iv = idx_v[pl.ds(base, 16)]
                    vals = plsc.load_gather(row_v, [iv])
                    pvec = lax.shift_right_logical(iota16 + base, _K.bit_length() - 1)
                    qv = plsc.load_gather(newp_v, [pvec])
                    out_v[pl.ds(base, 16)] = jnp.where(is_xyz, vals - qv, vals)
                    return carry

                lax.fori_loop(0, niter, body, 0)
                pltpu.sync_copy(out_v, nf_hbm.at[b, c])

                @pl.when(c < 3)
                def _():
                    pltpu.sync_copy(out_v, gx_hbm.at[b, c])

    return k(G, idx_flat, new_t)


def kernel(xyz, new_xyz, features):
    B, N, _ = xyz.shape
    P = new_xyz.shape[1]
    C = features.shape[1]
    xyz_t = jnp.transpose(xyz, (0, 2, 1))
    new_t = jnp.transpose(new_xyz, (0, 2, 1))
    idx = _knn_idx(new_xyz, xyz)

    G = jnp.concatenate([xyz_t, features], axis=1)
    nf, gx = _sc_gather(G, idx.reshape(B, P * _K), new_t)
    new_features = nf.reshape(B, 3 + C, P, _K)
    grouped_xyz = gx.reshape(B, 3, P, _K)
    return (new_features, grouped_xyz)
